# baseline (speedup 1.0000x reference)
import functools

import jax
import jax.numpy as jnp
from jax import lax
from jax.experimental import pallas as pl
from jax.experimental.pallas import tpu as pltpu
from jax.experimental.pallas import tpu_sc as plsc

_FP = 32


def _transpose_sc(V, D):
    info = plsc.get_sparse_core_info()
    NC, NS, L = info.num_cores, info.num_subcores, info.num_lanes
    NW = NC * NS
    NT = V // 128
    TAIL = V - NT * 128
    per_w = NT // NW
    extra = NT - per_w * NW
    WB = 8
    NB = ((per_w + 1 + WB - 1) // WB + 1) // 2 * 2
    NPAIR = NB // 2
    mesh = plsc.VectorSubcoreMesh(core_axis_name="c", subcore_axis_name="s")

    @functools.partial(
        pl.kernel,
        mesh=mesh,
        out_type=jax.ShapeDtypeStruct((V // 8, 128), jnp.float32),
        compiler_params=pltpu.CompilerParams(needs_layout_passes=False),
        scratch_types=[
            pltpu.VMEM((D, WB * 128), jnp.float32),
            pltpu.VMEM((D, WB * 128), jnp.float32),
            pltpu.VMEM((WB * 16, 128), jnp.float32),
            pltpu.VMEM((WB * 16, 128), jnp.float32),
            pltpu.SemaphoreType.DMA,
            pltpu.SemaphoreType.DMA,
            pltpu.SemaphoreType.DMA,
            pltpu.SemaphoreType.DMA,
        ],
    )
    def tr(tt_hbm, tail_hbm, out_hbm,
           blk0, blk1, out0, out1, si0, si1, so0, so1):
        wid = lax.axis_index("s") * NC + lax.axis_index("c")
        lane = lax.iota(jnp.int32, L)
        start = wid * per_w + jnp.minimum(wid, extra)
        end = start + per_w + jnp.where(wid < extra, 1, 0)

        def cp_in(k, blk, sem):
            bs = jnp.minimum(start + k * WB, end - WB)
            return pltpu.make_async_copy(
                tt_hbm.at[:, pl.ds(bs * 128, WB * 128)], blk, sem
            )

        def cp_out(k, out_v, sem):
            bs = jnp.minimum(start + k * WB, end - WB)
            return pltpu.make_async_copy(
                out_v, out_hbm.at[pl.ds(bs * 16, WB * 16)], sem
            )

        cp_in(0, blk0, si0).start()
        cp_in(1, blk1, si1).start()

        def pair_body(kk, carry):
            for phase, blk, out_v, si, so in (
                (0, blk0, out0, si0, so0),
                (1, blk1, out1, si1, so1),
            ):
                k = 2 * kk + phase
                cp_in(k, blk, si).wait()

                @pl.when(kk > 0)
                def _():
                    cp_out(k - 2, out_v, so).wait()

                rowp = lane // 8
                colp = (lane % 8) * L

                @plsc.parallel_loop(0, WB * 8, unroll=8)
                def col_body(k):
                    rbase = rowp + 2 * k
                    for d in range(D):
                        v = blk[d, pl.ds(k * L, L)]
                        plsc.store_scatter(out_v, [rbase, colp + d], v)

                cp_out(k, out_v, so).start()

                @pl.when(kk < NPAIR - 1)
                def _():
                    cp_in(k + 2, blk, si).start()
            return carry

        lax.fori_loop(0, NPAIR, pair_body, 0)
        cp_out(NB - 2, out0, so0).wait()
        cp_out(NB - 1, out1, so1).wait()

        if TAIL:
            @pl.when(wid == NW - 1)
            def _():
                pltpu.sync_copy(tail_hbm, out_hbm.at[pl.ds(NT * 16, TAIL // 8)])

    return tr


def _fm_sc(B, F, D):
    info = plsc.get_sparse_core_info()
    NC, NS, L = info.num_cores, info.num_subcores, info.num_lanes
    NW = NC * NS
    assert D == L and B % NW == 0
    b_per_w = B // NW
    C = 8
    n_chunks = b_per_w // C
    NPAIR = n_chunks // 2
    CF = C * F
    FP = _FP

    mesh = plsc.VectorSubcoreMesh(core_axis_name="c", subcore_axis_name="s")

    @functools.partial(
        pl.kernel,
        mesh=mesh,
        out_type=jax.ShapeDtypeStruct((B,), jnp.float32),
        compiler_params=pltpu.CompilerParams(needs_layout_passes=False),
        scratch_types=[
            pltpu.VMEM((b_per_w * F,), jnp.int32),
            pltpu.VMEM((b_per_w * FP,), jnp.float32),
            pltpu.VMEM((b_per_w * FP,), jnp.int32),
            pltpu.VMEM((CF, 128), jnp.float32),
            pltpu.VMEM((CF, 128), jnp.float32),
            pltpu.VMEM((b_per_w * D,), jnp.float32),
            pltpu.VMEM((b_per_w,), jnp.float32),
            pltpu.SemaphoreType.DMA,
            pltpu.SemaphoreType.DMA,
        ],
    )
    def fm(table_hbm, idxh_hbm, idxlo_hbm, vals_hbm, out_hbm,
           idxh_v, vals_v, idxlo_v, rows0, rows1, diffs_v, out_v, sg0, sg1):
        wid = lax.axis_index("s") * NC + lax.axis_index("c")
        lane = lax.iota(jnp.int32, L)
        base_s = wid * b_per_w

        pltpu.sync_copy(idxh_hbm.at[pl.ds(base_s * F, b_per_w * F)], idxh_v)
        pltpu.sync_copy(idxlo_hbm.at[pl.ds(base_s * FP, b_per_w * FP)], idxlo_v)
        pltpu.sync_copy(vals_hbm.at[pl.ds(base_s * FP, b_per_w * FP)], vals_v)

        def cp_g(j, rows, sem):
            return pltpu.make_async_copy(
                table_hbm.at[idxh_v.at[pl.ds(j * CF, CF)]], rows, sem
            )

        cp_g(0, rows0, sg0).start()
        cp_g(1, rows1, sg1).start()

        def pair_body(kk, carry):
            for phase, rows, sg in ((0, rows0, sg0), (1, rows1, sg1)):
                j = 2 * kk + phase
                cp_g(j, rows, sg).wait()

                @plsc.parallel_loop(0, C, unroll=2)
                def sample_body(b):
                    bg = j * C + b
                    p0 = b * F
                    v0 = vals_v[pl.ds(bg * FP, L)]
                    v1 = vals_v[pl.ds(bg * FP + L, L)]
                    o0 = idxlo_v[pl.ds(bg * FP, L)]
                    o1 = idxlo_v[pl.ds(bg * FP + L, L)]
                    acc = jnp.zeros((L,), jnp.float32)
                    acc2 = jnp.zeros((L,), jnp.float32)
                    for f in range(F):
                        vf = v0[f] if f < L else v1[f - L]
                        of = o0[f] if f < L else o1[f - L]
                        row = rows[p0 + f, pl.ds(of, L)]
                        w = row * vf
                        acc = acc + w
                        acc2 = acc2 + w * w
                    diffs_v[pl.ds(bg * D, D)] = acc * acc - acc2

                @pl.when(kk < NPAIR - 1)
                def _():
                    cp_g(j + 2, rows, sg).start()
            return carry

        lax.fori_loop(0, NPAIR, pair_body, 0)

        @plsc.parallel_loop(0, b_per_w // L, unroll=2)
        def group_body(g):
            row = (g * L + lane) * D
            tot = jnp.zeros((L,), jnp.float32)
            for d in range(D):
                tot = tot + plsc.load_gather(diffs_v, [row + d])
            out_v[pl.ds(g * L, L)] = 0.5 * tot

        pltpu.sync_copy(out_v, out_hbm.at[pl.ds(base_s, b_per_w)])

    return fm


def kernel(feature_indices, feature_values, embedding_weight):
    B, F = feature_indices.shape
    V, D = embedding_weight.shape
    tail = embedding_weight[(V // 128) * 128:].reshape(-1, 128)
    table128 = _transpose_sc(V, D)(embedding_weight.T, tail)
    idx_flat = feature_indices.reshape(B * F).astype(jnp.int32)
    idx_hi = idx_flat >> 3
    idx_lo = jnp.pad(
        (feature_indices.astype(jnp.int32) & 7) << 4, ((0, 0), (0, _FP - F))
    ).reshape(B * _FP)
    vals_pad = jnp.pad(feature_values, ((0, 0), (0, _FP - F))).reshape(B * _FP)
    out = _fm_sc(B, F, D)(table128, idx_hi, idx_lo, vals_pad)
    return out.reshape(B, 1)


# speedup vs baseline: 1.0618x; 1.0618x over previous
import functools

import jax
import jax.numpy as jnp
from jax import lax
from jax.experimental import pallas as pl
from jax.experimental.pallas import tpu as pltpu
from jax.experimental.pallas import tpu_sc as plsc

_FP = 32


def _transpose_sc(V, D):
    info = plsc.get_sparse_core_info()
    NC, NS, L = info.num_cores, info.num_subcores, info.num_lanes
    NW = NC * NS
    NT = V // 128
    TAIL = V - NT * 128
    per_w = NT // NW
    extra = NT - per_w * NW
    WB = 8
    NB = ((per_w + 1 + WB - 1) // WB + 1) // 2 * 2
    NPAIR = NB // 2
    mesh = plsc.VectorSubcoreMesh(core_axis_name="c", subcore_axis_name="s")

    @functools.partial(
        pl.kernel,
        mesh=mesh,
        out_type=jax.ShapeDtypeStruct((V // 8, 128), jnp.float32),
        compiler_params=pltpu.CompilerParams(needs_layout_passes=False),
        scratch_types=[
            pltpu.VMEM((D, WB * 128), jnp.float32),
            pltpu.VMEM((D, WB * 128), jnp.float32),
            pltpu.VMEM((WB * 16, 128), jnp.float32),
            pltpu.VMEM((WB * 16, 128), jnp.float32),
            pltpu.SemaphoreType.DMA,
            pltpu.SemaphoreType.DMA,
            pltpu.SemaphoreType.DMA,
            pltpu.SemaphoreType.DMA,
        ],
    )
    def tr(tt_hbm, tail_hbm, out_hbm,
           blk0, blk1, out0, out1, si0, si1, so0, so1):
        wid = lax.axis_index("s") * NC + lax.axis_index("c")
        lane = lax.iota(jnp.int32, L)
        start = wid * per_w + jnp.minimum(wid, extra)
        end = start + per_w + jnp.where(wid < extra, 1, 0)

        def cp_in(k, blk, sem):
            bs = jnp.minimum(start + k * WB, end - WB)
            return pltpu.make_async_copy(
                tt_hbm.at[:, pl.ds(bs * 128, WB * 128)], blk, sem
            )

        def cp_out(k, out_v, sem):
            bs = jnp.minimum(start + k * WB, end - WB)
            return pltpu.make_async_copy(
                out_v, out_hbm.at[pl.ds(bs * 16, WB * 16)], sem
            )

        cp_in(0, blk0, si0).start()
        cp_in(1, blk1, si1).start()

        def pair_body(kk, carry):
            for phase, blk, out_v, si, so in (
                (0, blk0, out0, si0, so0),
                (1, blk1, out1, si1, so1),
            ):
                k = 2 * kk + phase
                cp_in(k, blk, si).wait()

                @pl.when(kk > 0)
                def _():
                    cp_out(k - 2, out_v, so).wait()

                rowp = lane // 8
                colp = (lane % 8) * L

                @plsc.parallel_loop(0, WB * 8, unroll=4)
                def col_body(k):
                    rbase = rowp + 2 * k
                    for d in range(D):
                        v = blk[d, pl.ds(k * L, L)]
                        plsc.store_scatter(out_v, [rbase, colp + d], v)

                cp_out(k, out_v, so).start()

                @pl.when(kk < NPAIR - 1)
                def _():
                    cp_in(k + 2, blk, si).start()
            return carry

        lax.fori_loop(0, NPAIR, pair_body, 0)
        cp_out(NB - 2, out0, so0).wait()
        cp_out(NB - 1, out1, so1).wait()

        if TAIL:
            @pl.when(wid == NW - 1)
            def _():
                pltpu.sync_copy(tail_hbm, out_hbm.at[pl.ds(NT * 16, TAIL // 8)])

    return tr


def _fm_sc(B, F, D):
    info = plsc.get_sparse_core_info()
    NC, NS, L = info.num_cores, info.num_subcores, info.num_lanes
    NW = NC * NS
    assert D == L and B % NW == 0
    b_per_w = B // NW
    C = 8
    n_chunks = b_per_w // C
    NPAIR = n_chunks // 2
    CF = C * F
    FP = _FP

    mesh = plsc.VectorSubcoreMesh(core_axis_name="c", subcore_axis_name="s")

    @functools.partial(
        pl.kernel,
        mesh=mesh,
        out_type=jax.ShapeDtypeStruct((B,), jnp.float32),
        compiler_params=pltpu.CompilerParams(needs_layout_passes=False),
        scratch_types=[
            pltpu.VMEM((b_per_w * F,), jnp.int32),
            pltpu.VMEM((b_per_w * FP,), jnp.float32),
            pltpu.VMEM((b_per_w * FP,), jnp.int32),
            pltpu.VMEM((CF, 128), jnp.float32),
            pltpu.VMEM((CF, 128), jnp.float32),
            pltpu.VMEM((b_per_w * D,), jnp.float32),
            pltpu.VMEM((b_per_w,), jnp.float32),
            pltpu.SemaphoreType.DMA,
            pltpu.SemaphoreType.DMA,
        ],
    )
    def fm(table_hbm, idxh_hbm, idxlo_hbm, vals_hbm, out_hbm,
           idxh_v, vals_v, idxlo_v, rows0, rows1, diffs_v, out_v, sg0, sg1):
        wid = lax.axis_index("s") * NC + lax.axis_index("c")
        lane = lax.iota(jnp.int32, L)
        base_s = wid * b_per_w

        pltpu.sync_copy(idxh_hbm.at[pl.ds(base_s * F, b_per_w * F)], idxh_v)
        pltpu.sync_copy(idxlo_hbm.at[pl.ds(base_s * FP, b_per_w * FP)], idxlo_v)
        pltpu.sync_copy(vals_hbm.at[pl.ds(base_s * FP, b_per_w * FP)], vals_v)

        def cp_g(j, rows, sem):
            return pltpu.make_async_copy(
                table_hbm.at[idxh_v.at[pl.ds(j * CF, CF)]], rows, sem
            )

        cp_g(0, rows0, sg0).start()
        cp_g(1, rows1, sg1).start()

        def pair_body(kk, carry):
            for phase, rows, sg in ((0, rows0, sg0), (1, rows1, sg1)):
                j = 2 * kk + phase
                cp_g(j, rows, sg).wait()

                @plsc.parallel_loop(0, C, unroll=2)
                def sample_body(b):
                    bg = j * C + b
                    p0 = b * F
                    v0 = vals_v[pl.ds(bg * FP, L)]
                    v1 = vals_v[pl.ds(bg * FP + L, L)]
                    o0 = idxlo_v[pl.ds(bg * FP, L)]
                    o1 = idxlo_v[pl.ds(bg * FP + L, L)]
                    acc = jnp.zeros((L,), jnp.float32)
                    acc2 = jnp.zeros((L,), jnp.float32)
                    for f in range(F):
                        vf = v0[f] if f < L else v1[f - L]
                        of = o0[f] if f < L else o1[f - L]
                        row = rows[p0 + f, pl.ds(of, L)]
                        w = row * vf
                        acc = acc + w
                        acc2 = acc2 + w * w
                    diffs_v[pl.ds(bg * D, D)] = acc * acc - acc2

                @pl.when(kk < NPAIR - 1)
                def _():
                    cp_g(j + 2, rows, sg).start()
            return carry

        lax.fori_loop(0, NPAIR, pair_body, 0)

        @plsc.parallel_loop(0, b_per_w // L, unroll=2)
        def group_body(g):
            row = (g * L + lane) * D
            tot = jnp.zeros((L,), jnp.float32)
            for d in range(D):
                tot = tot + plsc.load_gather(diffs_v, [row + d])
            out_v[pl.ds(g * L, L)] = 0.5 * tot

        pltpu.sync_copy(out_v, out_hbm.at[pl.ds(base_s, b_per_w)])

    return fm


def kernel(feature_indices, feature_values, embedding_weight):
    B, F = feature_indices.shape
    V, D = embedding_weight.shape
    tail = embedding_weight[(V // 128) * 128:].reshape(-1, 128)
    table128 = _transpose_sc(V, D)(embedding_weight.T, tail)
    idx_flat = feature_indices.reshape(B * F).astype(jnp.int32)
    idx_hi = idx_flat >> 3
    idx_lo = jnp.pad(
        (feature_indices.astype(jnp.int32) & 7) << 4, ((0, 0), (0, _FP - F))
    ).reshape(B * _FP)
    vals_pad = jnp.pad(feature_values, ((0, 0), (0, _FP - F))).reshape(B * _FP)
    out = _fm_sc(B, F, D)(table128, idx_hi, idx_lo, vals_pad)
    return out.reshape(B, 1)
